# Initial kernel scaffold; baseline (speedup 1.0000x reference)
import jax
import jax.numpy as jnp
from jax.experimental import pallas as pl


def kernel(cat_index, cat_val, field_size, table):
    raise NotImplementedError("write your pallas kernel here")


# speedup vs baseline: 1.3485x; 1.3485x over previous
import functools

import jax
import jax.numpy as jnp
from jax import lax
from jax.experimental import pallas as pl
from jax.experimental.pallas import tpu as pltpu
from jax.experimental.pallas import tpu_sc as plsc

_NC = 2
_NS = 16
_NW = _NC * _NS

_GROUP = 128
_CHUNK = 1024


@functools.cache
def _build(N, D):
    n_per_w = N // _NW
    g_per_w = n_per_w // _GROUP
    c_per_w = n_per_w // _CHUNK
    g_per_c = _CHUNK // _GROUP
    mesh = plsc.VectorSubcoreMesh(core_axis_name="c", subcore_axis_name="s")

    @functools.partial(
        pl.kernel,
        mesh=mesh,
        out_type=jax.ShapeDtypeStruct((N, D), jnp.float32),
        compiler_params=pltpu.CompilerParams(use_tc_tiling_on_sc=False),
        scratch_types=[
            pltpu.VMEM((g_per_w, _GROUP), jnp.int32),
            pltpu.VMEM((n_per_w,), jnp.float32),
            pltpu.VMEM((_CHUNK, D), jnp.float32),
            pltpu.SemaphoreType.DMA,
        ],
    )
    def k(idx_hbm, val_hbm, table_hbm, out_hbm, idx_v, val_v, rows_v, sem):
        wid = lax.axis_index("s") * _NC + lax.axis_index("c")
        gbase = wid * g_per_w
        rbase = wid * n_per_w
        pltpu.sync_copy(idx_hbm.at[pl.ds(gbase, g_per_w)], idx_v)
        pltpu.sync_copy(val_hbm.at[pl.ds(rbase, n_per_w)], val_v)

        def chunk_body(c, carry):
            row0 = c * _CHUNK
            copies = []
            for j in range(g_per_c):
                copies.append(pltpu.async_copy(
                    table_hbm.at[idx_v.at[c * g_per_c + j]],
                    rows_v.at[pl.ds(j * _GROUP, _GROUP)],
                    sem,
                ))
            for cp in copies:
                cp.wait()

            def mul_body(i, carry2):
                vvec = val_v[pl.ds(row0 + i * 16, 16)]
                for u in range(16):
                    r = i * 16 + u
                    v = vvec[u]
                    for h in range(D // 16):
                        rows_v[r, pl.ds(h * 16, 16)] = (
                            rows_v[r, pl.ds(h * 16, 16)] * v)
                return carry2

            lax.fori_loop(0, _CHUNK // 16, mul_body, 0)
            pltpu.sync_copy(rows_v, out_hbm.at[pl.ds(rbase + row0, _CHUNK)])
            return carry

        lax.fori_loop(0, c_per_w, chunk_body, 0)

    return k


def kernel(cat_index, cat_val, field_size, table):
    B, F = cat_index.shape
    D = table.shape[1]
    N = B * F
    idx2d = cat_index.reshape(N // _GROUP, _GROUP).astype(jnp.int32)
    val = cat_val.reshape(N)
    out = _build(N, D)(idx2d, val, table)
    return out.reshape(B, F, D)
